# baseline (speedup 1.0000x reference)
import functools

import jax
import jax.numpy as jnp
from jax import lax
from jax.experimental import pallas as pl
from jax.experimental.pallas import tpu as pltpu
from jax.experimental.pallas import tpu_sc as plsc

_NC = 2
_NS = 16
_NW = _NC * _NS


@functools.lru_cache(maxsize=None)
def _make_sc(sc_rows, batch, embed_dim, dtype):
    rows_per_w = sc_rows // _NW
    chunk = min(rows_per_w, 128)
    n_chunks = rows_per_w // chunk
    mesh = plsc.VectorSubcoreMesh(core_axis_name="c", subcore_axis_name="s")

    @functools.partial(
        pl.kernel,
        mesh=mesh,
        out_type=jax.ShapeDtypeStruct((sc_rows, batch, embed_dim), dtype),
        scratch_types=[
            pltpu.VMEM((chunk, embed_dim), dtype),
            pltpu.SemaphoreType.DMA,
            pltpu.SemaphoreType.DMA,
        ],
    )
    def k(table_hbm, out_hbm, buf, lsem, wsem):
        wid = lax.axis_index("s") * _NC + lax.axis_index("c")
        base = wid * rows_per_w
        for c in range(n_chunks):
            s0 = base + c * chunk
            pltpu.make_async_copy(
                table_hbm.at[pl.ds(s0, chunk)], buf, lsem
            ).start()
            pltpu.make_async_copy(
                table_hbm.at[pl.ds(s0, chunk)], buf, lsem
            ).wait()
            ws = []
            for b in range(batch):
                d = pltpu.make_async_copy(
                    buf, out_hbm.at[pl.ds(s0, chunk), b], wsem
                )
                d.start()
                ws.append(d)
            for d in ws:
                d.wait()

    return k


def _tc_body(emb_ref, out_ref):
    emb = emb_ref[...]
    out_ref[...] = jnp.broadcast_to(emb[:, None, :], out_ref.shape)


def kernel(x, pos_embedding):
    seq_len, batch = x.shape
    max_len, embed_dim = pos_embedding.shape
    sc_rows = seq_len // 4
    sc_out = _make_sc(sc_rows, batch, embed_dim, pos_embedding.dtype)(
        pos_embedding
    )

    blk = 512
    off = sc_rows // blk
    tc_out = pl.pallas_call(
        _tc_body,
        grid=((seq_len - sc_rows) // blk,),
        in_specs=[pl.BlockSpec((blk, embed_dim), lambda i, o=off: (i + o, 0))],
        out_specs=pl.BlockSpec(
            (blk, batch, embed_dim), lambda i, o=off: (i + o, 0, 0)
        ),
        out_shape=jax.ShapeDtypeStruct(
            (seq_len, batch, embed_dim), pos_embedding.dtype
        ),
    )(pos_embedding)

    return lax.dynamic_update_slice(tc_out, sc_out, (0, 0, 0))


# speedup vs baseline: 1.2976x; 1.2976x over previous
import functools

import jax
from jax import lax
from jax.experimental import pallas as pl
from jax.experimental.pallas import tpu as pltpu
from jax.experimental.pallas import tpu_sc as plsc

_NC = 2
_NS = 16
_NW = _NC * _NS


@functools.lru_cache(maxsize=None)
def _make_sc(seq_len, batch, embed_dim, dtype):
    rows_per_w = seq_len // _NW
    chunk = min(rows_per_w, 128)
    n_chunks = rows_per_w // chunk
    mesh = plsc.VectorSubcoreMesh(core_axis_name="c", subcore_axis_name="s")

    @functools.partial(
        pl.kernel,
        mesh=mesh,
        out_type=jax.ShapeDtypeStruct((seq_len, batch, embed_dim), dtype),
        scratch_types=[
            pltpu.VMEM((chunk, embed_dim), dtype),
            pltpu.SemaphoreType.DMA,
            pltpu.SemaphoreType.DMA,
        ],
    )
    def k(table_hbm, out_hbm, buf, lsem, wsem):
        wid = lax.axis_index("s") * _NC + lax.axis_index("c")
        base = wid * rows_per_w
        for c in range(n_chunks):
            s0 = base + c * chunk
            ld = pltpu.make_async_copy(
                table_hbm.at[pl.ds(s0, chunk)], buf, lsem
            )
            ld.start()
            ld.wait()
            ws = []
            for b in range(batch):
                d = pltpu.make_async_copy(
                    buf, out_hbm.at[pl.ds(s0, chunk), b], wsem
                )
                d.start()
                ws.append(d)
            for d in ws:
                d.wait()

    return k


def kernel(x, pos_embedding):
    seq_len, batch = x.shape
    max_len, embed_dim = pos_embedding.shape
    k = _make_sc(seq_len, batch, embed_dim, pos_embedding.dtype)
    return k(pos_embedding)
